# Initial kernel scaffold; baseline (speedup 1.0000x reference)
import jax
import jax.numpy as jnp
from jax.experimental import pallas as pl


def kernel(x, edge_index, examples, W1, b1, W2, b2, Wfc, bfc):
    raise NotImplementedError("write your pallas kernel here")


# speedup vs baseline: 26.9964x; 26.9964x over previous
import functools

import jax
import jax.numpy as jnp
from jax import lax
from jax.experimental import pallas as pl
from jax.experimental.pallas import tpu as pltpu
from jax.experimental.pallas import tpu_sc as plsc

N = 10000
E = 320000
NEX = 100000
NC, NS, L = 2, 16, 16
NW = NC * NS

CH = 128
NCH = 80
E_PAD = NW * NCH * CH
DUMMY = N
ACC_ROWS = 10112
RPS = ACC_ROWS // NS

EX_T = 3200
EX_PAD = NW * EX_T

def _f32(*shape):
    return jax.ShapeDtypeStruct(shape, jnp.float32)


@functools.cache
def _mesh():
    return plsc.VectorSubcoreMesh(
        core_axis_name="c", subcore_axis_name="s",
        num_cores=NC, num_subcores=NS)


_SC_PARAMS = pltpu.CompilerParams(
    use_tc_tiling_on_sc=False, needs_layout_passes=False)


@functools.cache
def _make_sc_degree():
    @functools.partial(
        pl.kernel,
        out_type=[_f32(ACC_ROWS, 16), _f32(ACC_ROWS, 16)],
        mesh=_mesh(),
        compiler_params=_SC_PARAMS,
        scratch_types=[
            pltpu.VMEM((NCH, CH), jnp.int32),
            pltpu.VMEM((CH, 16), jnp.float32),
            pltpu.VMEM_SHARED((ACC_ROWS, 16), jnp.float32),
        ],
    )
    def k(dst_hbm, ones_hbm, zeros_hbm, p0_hbm, p1_hbm, dstv, onesv, acc):
        cid = lax.axis_index("c")
        sid = lax.axis_index("s")
        wid = cid * NS + sid
        rs = pl.ds(sid * RPS, RPS)
        pltpu.sync_copy(dst_hbm.at[wid], dstv)
        pltpu.sync_copy(ones_hbm, onesv)
        pltpu.sync_copy(zeros_hbm.at[rs], acc.at[rs])
        plsc.subcore_barrier()

        @pl.loop(0, NCH)
        def _(c):
            pltpu.sync_copy(onesv, acc.at[dstv.at[c]], add=True)

        plsc.subcore_barrier()

        @pl.when(cid == 0)
        def _():
            pltpu.sync_copy(acc.at[rs], p0_hbm.at[rs])

        @pl.when(cid == 1)
        def _():
            pltpu.sync_copy(acc.at[rs], p1_hbm.at[rs])

    return k


@functools.cache
def _make_sc_aggregate(F):
    @functools.partial(
        pl.kernel,
        out_type=[_f32(ACC_ROWS, F), _f32(ACC_ROWS, F)],
        mesh=_mesh(),
        compiler_params=_SC_PARAMS,
        scratch_types=[
            pltpu.VMEM((NCH, CH), jnp.int32),
            pltpu.VMEM((NCH, CH), jnp.int32),
            pltpu.VMEM((CH, F), jnp.float32),
            pltpu.VMEM((CH, F), jnp.float32),
            pltpu.VMEM_SHARED((ACC_ROWS, F), jnp.float32),
            pltpu.SemaphoreType.DMA,
            pltpu.SemaphoreType.DMA,
        ],
    )
    def k(src_hbm, dst_hbm, z_hbm, zeros_hbm, p0_hbm, p1_hbm,
          srcv, dstv, bufa, bufb, acc, sema, semb):
        cid = lax.axis_index("c")
        sid = lax.axis_index("s")
        wid = cid * NS + sid
        rs = pl.ds(sid * RPS, RPS)
        pltpu.sync_copy(src_hbm.at[wid], srcv)
        pltpu.sync_copy(dst_hbm.at[wid], dstv)
        pltpu.sync_copy(zeros_hbm.at[rs], acc.at[rs])
        plsc.subcore_barrier()

        @pl.loop(0, NCH, step=2)
        def _(c):
            ga = pltpu.async_copy(z_hbm.at[srcv.at[c]], bufa, sema)
            gb = pltpu.async_copy(z_hbm.at[srcv.at[c + 1]], bufb, semb)
            ga.wait()
            pltpu.sync_copy(bufa, acc.at[dstv.at[c]], add=True)
            gb.wait()
            pltpu.sync_copy(bufb, acc.at[dstv.at[c + 1]], add=True)

        plsc.subcore_barrier()

        @pl.when(cid == 0)
        def _():
            pltpu.sync_copy(acc.at[rs], p0_hbm.at[rs])

        @pl.when(cid == 1)
        def _():
            pltpu.sync_copy(acc.at[rs], p1_hbm.at[rs])

    return k


@functools.cache
def _make_sc_score():
    @functools.partial(
        pl.kernel,
        out_type=_f32(EX_PAD),
        mesh=_mesh(),
        compiler_params=_SC_PARAMS,
        scratch_types=[
            pltpu.VMEM((N,), jnp.float32),
            pltpu.VMEM((N,), jnp.float32),
            pltpu.VMEM((EX_T,), jnp.int32),
            pltpu.VMEM((EX_T,), jnp.int32),
            pltpu.VMEM((EX_T,), jnp.float32),
        ],
    )
    def k(scores_hbm, ex0_hbm, ex1_hbm, out_hbm, siv, sjv, e0v, e1v, outv):
        cid = lax.axis_index("c")
        sid = lax.axis_index("s")
        wid = cid * NS + sid
        pltpu.sync_copy(scores_hbm.at[0], siv)
        pltpu.sync_copy(scores_hbm.at[1], sjv)
        pltpu.sync_copy(ex0_hbm.at[wid], e0v)
        pltpu.sync_copy(ex1_hbm.at[wid], e1v)

        @pl.loop(0, EX_T, step=L)
        def _(i):
            i0 = e0v.at[pl.ds(i, L)][...]
            i1 = e1v.at[pl.ds(i, L)][...]
            a = plsc.load_gather(siv, [i0])
            b = plsc.load_gather(sjv, [i1])
            outv.at[pl.ds(i, L)][...] = 1.0 / (1.0 + jnp.exp(-(a + b)))

        pltpu.sync_copy(outv, out_hbm.at[pl.ds(wid * EX_T, EX_T)])

    return k


def _tc1_body(x_ref, w1_ref, d0_ref, d1_ref, z1_ref, y_ref):
    deg = d0_ref[0:N, :] + d1_ref[0:N, :] + 1.0
    y16 = lax.rsqrt(deg)
    xw = jnp.dot(x_ref[...], w1_ref[...], preferred_element_type=jnp.float32,
                 precision=lax.Precision.HIGHEST)
    z1_ref[...] = xw * y16
    y_ref[...] = y16


def _tc1(x, w1, d0, d1):
    return pl.pallas_call(
        _tc1_body, out_shape=[_f32(N, 16), _f32(N, 16)]
    )(x, w1, d0, d1)


def _tc2_body(y_ref, z1_ref, p0_ref, p1_ref, b1_ref, w2_ref, z2_ref):
    y16 = y_ref[...]
    h1 = jnp.maximum(y16 * (p0_ref[0:N, :] + p1_ref[0:N, :] + z1_ref[...])
                     + b1_ref[...], 0.0)
    xw2 = jnp.dot(h1, w2_ref[...], preferred_element_type=jnp.float32,
                  precision=lax.Precision.HIGHEST)
    y32 = jnp.concatenate([y16, y16], axis=1)
    z2_ref[...] = xw2 * y32


def _tc2(y16, z1, p0, p1, b1, w2):
    return pl.pallas_call(_tc2_body, out_shape=_f32(N, 32))(
        y16, z1, p0, p1, b1, w2)


def _tc3_body(y_ref, z2_ref, q0_ref, q1_ref, b2_ref, wfc_ref, bfc_ref,
              scores_ref):
    y16 = y_ref[...]
    y32 = jnp.concatenate([y16, y16], axis=1)
    h2 = y32 * (q0_ref[0:N, :] + q1_ref[0:N, :] + z2_ref[...]) + b2_ref[...]
    wi = wfc_ref[0:32, :]
    wj = wfc_ref[32:64, :]
    dn = (((0,), (1,)), ((), ()))
    si = lax.dot_general(wi, h2, dn, preferred_element_type=jnp.float32,
                         precision=lax.Precision.HIGHEST)
    sj = lax.dot_general(wj, h2, dn, preferred_element_type=jnp.float32,
                         precision=lax.Precision.HIGHEST)
    scores_ref[0:1, :] = si + bfc_ref[...]
    scores_ref[1:2, :] = sj


def _tc3(y16, z2, q0, q1, b2, wfc, bfc):
    return pl.pallas_call(_tc3_body, out_shape=_f32(2, N))(
        y16, z2, q0, q1, b2, wfc, bfc)


def kernel(x, edge_index, examples, W1, b1, W2, b2, Wfc, bfc):
    src = edge_index[0].astype(jnp.int32)
    dst = edge_index[1].astype(jnp.int32)
    epad = E_PAD - E
    src_p = jnp.concatenate(
        [src, jnp.zeros((epad,), jnp.int32)]).reshape(NW, NCH, CH)
    dst_p = jnp.concatenate(
        [dst, jnp.full((epad,), DUMMY, jnp.int32)]).reshape(NW, NCH, CH)
    xpad = EX_PAD - NEX
    ex0 = jnp.concatenate(
        [examples[:, 0].astype(jnp.int32), jnp.zeros((xpad,), jnp.int32)]
    ).reshape(NW, EX_T)
    ex1 = jnp.concatenate(
        [examples[:, 1].astype(jnp.int32), jnp.zeros((xpad,), jnp.int32)]
    ).reshape(NW, EX_T)

    ones16 = jnp.ones((CH, 16), jnp.float32)
    zeros16 = jnp.zeros((ACC_ROWS, 16), jnp.float32)
    zeros32 = jnp.zeros((ACC_ROWS, 32), jnp.float32)

    d0, d1 = _make_sc_degree()(dst_p, ones16, zeros16)
    z1, y16 = _tc1(x, W1, d0, d1)
    p0, p1 = _make_sc_aggregate(16)(src_p, dst_p, z1, zeros16)
    z2 = _tc2(y16, z1, p0, p1, b1.reshape(1, 16), W2)
    q0, q1 = _make_sc_aggregate(32)(src_p, dst_p, z2, zeros32)
    scores = _tc3(y16, z2, q0, q1, b2.reshape(1, 32), Wfc,
                  bfc.reshape(1, 1))
    out = _make_sc_score()(scores, ex0, ex1)
    return out[:NEX]


# speedup vs baseline: 42.3725x; 1.5696x over previous
import functools

import jax
import jax.numpy as jnp
from jax import lax
from jax.experimental import pallas as pl
from jax.experimental.pallas import tpu as pltpu
from jax.experimental.pallas import tpu_sc as plsc

N = 10000
E = 320000
NEX = 100000
NC, NS, L = 2, 16, 16
NW = NC * NS

CH = 128
K = 8
NCH = 10
E_PAD = NW * NCH * K * CH
Z_SH_ROWS = 10240
DUMMY = N
ACC_ROWS = 10112
RPS = ACC_ROWS // NS

EX_T = 3200
EX_PAD = NW * EX_T

def _f32(*shape):
    return jax.ShapeDtypeStruct(shape, jnp.float32)


@functools.cache
def _mesh():
    return plsc.VectorSubcoreMesh(
        core_axis_name="c", subcore_axis_name="s",
        num_cores=NC, num_subcores=NS)


_SC_PARAMS = pltpu.CompilerParams(
    use_tc_tiling_on_sc=False, needs_layout_passes=False)


@functools.cache
def _make_sc_degree():
    @functools.partial(
        pl.kernel,
        out_type=[_f32(ACC_ROWS, 16), _f32(ACC_ROWS, 16)],
        mesh=_mesh(),
        compiler_params=_SC_PARAMS,
        scratch_types=[
            pltpu.VMEM((NCH, K * CH), jnp.int32),
            pltpu.VMEM((K * CH, 16), jnp.float32),
            pltpu.VMEM_SHARED((ACC_ROWS, 16), jnp.float32),
        ],
    )
    def k(dst_hbm, ones_hbm, zeros_hbm, p0_hbm, p1_hbm, dstv, onesv, acc):
        cid = lax.axis_index("c")
        sid = lax.axis_index("s")
        wid = cid * NS + sid
        rs = pl.ds(sid * RPS, RPS)
        pltpu.sync_copy(dst_hbm.at[wid], dstv)
        pltpu.sync_copy(ones_hbm, onesv)
        pltpu.sync_copy(zeros_hbm.at[rs], acc.at[rs])
        plsc.subcore_barrier()

        @pl.loop(0, NCH)
        def _(c):
            pltpu.sync_copy(onesv, acc.at[dstv.at[c]], add=True)

        plsc.subcore_barrier()

        @pl.when(cid == 0)
        def _():
            pltpu.sync_copy(acc.at[rs], p0_hbm.at[rs])

        @pl.when(cid == 1)
        def _():
            pltpu.sync_copy(acc.at[rs], p1_hbm.at[rs])

    return k


@functools.cache
def _make_sc_aggregate(F):
    @functools.partial(
        pl.kernel,
        out_type=[_f32(ACC_ROWS, F), _f32(ACC_ROWS, F)],
        mesh=_mesh(),
        compiler_params=_SC_PARAMS,
        scratch_types=[
            pltpu.VMEM((NCH, K * CH), jnp.int32),
            pltpu.VMEM((NCH, K * CH), jnp.int32),
            pltpu.VMEM((K * CH, F), jnp.float32),
            pltpu.VMEM((K * CH, F), jnp.float32),
            pltpu.VMEM_SHARED((Z_SH_ROWS, F), jnp.float32),
            pltpu.VMEM_SHARED((ACC_ROWS, F), jnp.float32),
            pltpu.SemaphoreType.DMA,
            pltpu.SemaphoreType.DMA,
        ],
    )
    def k(src_hbm, dst_hbm, z_hbm, zeros_hbm, p0_hbm, p1_hbm,
          srcv, dstv, bufa, bufb, z_sh, acc, sema, semb):
        cid = lax.axis_index("c")
        sid = lax.axis_index("s")
        wid = cid * NS + sid
        rs = pl.ds(sid * RPS, RPS)
        pltpu.sync_copy(src_hbm.at[wid], srcv)
        pltpu.sync_copy(dst_hbm.at[wid], dstv)
        pltpu.sync_copy(zeros_hbm.at[rs], acc.at[rs])

        @pl.when(sid < NS - 1)
        def _():
            zs = pl.ds(sid * 640, 640)
            pltpu.sync_copy(z_hbm.at[zs], z_sh.at[zs])

        @pl.when(sid == NS - 1)
        def _():
            zs = pl.ds((NS - 1) * 640, 400)
            pltpu.sync_copy(z_hbm.at[zs], z_sh.at[zs])

        plsc.subcore_barrier()

        @pl.loop(0, NCH, step=2)
        def _(c):
            ga = pltpu.async_copy(z_sh.at[srcv.at[c]], bufa, sema)
            gb = pltpu.async_copy(z_sh.at[srcv.at[c + 1]], bufb, semb)
            ga.wait()
            pltpu.sync_copy(bufa, acc.at[dstv.at[c]], add=True)
            gb.wait()
            pltpu.sync_copy(bufb, acc.at[dstv.at[c + 1]], add=True)

        plsc.subcore_barrier()

        @pl.when(cid == 0)
        def _():
            pltpu.sync_copy(acc.at[rs], p0_hbm.at[rs])

        @pl.when(cid == 1)
        def _():
            pltpu.sync_copy(acc.at[rs], p1_hbm.at[rs])

    return k


@functools.cache
def _make_sc_score():
    @functools.partial(
        pl.kernel,
        out_type=_f32(EX_PAD),
        mesh=_mesh(),
        compiler_params=_SC_PARAMS,
        scratch_types=[
            pltpu.VMEM((N,), jnp.float32),
            pltpu.VMEM((N,), jnp.float32),
            pltpu.VMEM((EX_T,), jnp.int32),
            pltpu.VMEM((EX_T,), jnp.int32),
            pltpu.VMEM((EX_T,), jnp.float32),
        ],
    )
    def k(scores_hbm, ex0_hbm, ex1_hbm, out_hbm, siv, sjv, e0v, e1v, outv):
        cid = lax.axis_index("c")
        sid = lax.axis_index("s")
        wid = cid * NS + sid
        pltpu.sync_copy(scores_hbm.at[0], siv)
        pltpu.sync_copy(scores_hbm.at[1], sjv)
        pltpu.sync_copy(ex0_hbm.at[wid], e0v)
        pltpu.sync_copy(ex1_hbm.at[wid], e1v)

        @pl.loop(0, EX_T, step=L)
        def _(i):
            i0 = e0v.at[pl.ds(i, L)][...]
            i1 = e1v.at[pl.ds(i, L)][...]
            a = plsc.load_gather(siv, [i0])
            b = plsc.load_gather(sjv, [i1])
            outv.at[pl.ds(i, L)][...] = 1.0 / (1.0 + jnp.exp(-(a + b)))

        pltpu.sync_copy(outv, out_hbm.at[pl.ds(wid * EX_T, EX_T)])

    return k


def _tc1_body(x_ref, w1_ref, d0_ref, d1_ref, z1_ref, y_ref):
    deg = d0_ref[0:N, :] + d1_ref[0:N, :] + 1.0
    y16 = lax.rsqrt(deg)
    xw = jnp.dot(x_ref[...], w1_ref[...], preferred_element_type=jnp.float32,
                 precision=lax.Precision.HIGHEST)
    z1_ref[...] = xw * y16
    y_ref[...] = y16


def _tc1(x, w1, d0, d1):
    return pl.pallas_call(
        _tc1_body, out_shape=[_f32(N, 16), _f32(N, 16)]
    )(x, w1, d0, d1)


def _tc2_body(y_ref, z1_ref, p0_ref, p1_ref, b1_ref, w2_ref, z2_ref):
    y16 = y_ref[...]
    h1 = jnp.maximum(y16 * (p0_ref[0:N, :] + p1_ref[0:N, :] + z1_ref[...])
                     + b1_ref[...], 0.0)
    xw2 = jnp.dot(h1, w2_ref[...], preferred_element_type=jnp.float32,
                  precision=lax.Precision.HIGHEST)
    y32 = jnp.concatenate([y16, y16], axis=1)
    z2_ref[...] = xw2 * y32


def _tc2(y16, z1, p0, p1, b1, w2):
    return pl.pallas_call(_tc2_body, out_shape=_f32(N, 32))(
        y16, z1, p0, p1, b1, w2)


def _tc3_body(y_ref, z2_ref, q0_ref, q1_ref, b2_ref, wfc_ref, bfc_ref,
              scores_ref):
    y16 = y_ref[...]
    y32 = jnp.concatenate([y16, y16], axis=1)
    h2 = y32 * (q0_ref[0:N, :] + q1_ref[0:N, :] + z2_ref[...]) + b2_ref[...]
    wi = wfc_ref[0:32, :]
    wj = wfc_ref[32:64, :]
    dn = (((0,), (1,)), ((), ()))
    si = lax.dot_general(wi, h2, dn, preferred_element_type=jnp.float32,
                         precision=lax.Precision.HIGHEST)
    sj = lax.dot_general(wj, h2, dn, preferred_element_type=jnp.float32,
                         precision=lax.Precision.HIGHEST)
    scores_ref[0:1, :] = si + bfc_ref[...]
    scores_ref[1:2, :] = sj


def _tc3(y16, z2, q0, q1, b2, wfc, bfc):
    return pl.pallas_call(_tc3_body, out_shape=_f32(2, N))(
        y16, z2, q0, q1, b2, wfc, bfc)


def kernel(x, edge_index, examples, W1, b1, W2, b2, Wfc, bfc):
    src = edge_index[0].astype(jnp.int32)
    dst = edge_index[1].astype(jnp.int32)
    epad = E_PAD - E
    src_p = jnp.concatenate(
        [src, jnp.zeros((epad,), jnp.int32)]).reshape(NW, NCH, K * CH)
    dst_p = jnp.concatenate(
        [dst, jnp.full((epad,), DUMMY, jnp.int32)]).reshape(NW, NCH, K * CH)
    xpad = EX_PAD - NEX
    ex0 = jnp.concatenate(
        [examples[:, 0].astype(jnp.int32), jnp.zeros((xpad,), jnp.int32)]
    ).reshape(NW, EX_T)
    ex1 = jnp.concatenate(
        [examples[:, 1].astype(jnp.int32), jnp.zeros((xpad,), jnp.int32)]
    ).reshape(NW, EX_T)

    ones16 = jnp.ones((K * CH, 16), jnp.float32)
    zeros16 = jnp.zeros((ACC_ROWS, 16), jnp.float32)
    zeros32 = jnp.zeros((ACC_ROWS, 32), jnp.float32)

    d0, d1 = _make_sc_degree()(dst_p, ones16, zeros16)
    z1, y16 = _tc1(x, W1, d0, d1)
    p0, p1 = _make_sc_aggregate(16)(src_p, dst_p, z1, zeros16)
    z2 = _tc2(y16, z1, p0, p1, b1.reshape(1, 16), W2)
    q0, q1 = _make_sc_aggregate(32)(src_p, dst_p, z2, zeros32)
    scores = _tc3(y16, z2, q0, q1, b2.reshape(1, 32), Wfc,
                  bfc.reshape(1, 1))
    out = _make_sc_score()(scores, ex0, ex1)
    return out[:NEX]


# speedup vs baseline: 46.7224x; 1.1027x over previous
import functools

import jax
import jax.numpy as jnp
from jax import lax
from jax.experimental import pallas as pl
from jax.experimental.pallas import tpu as pltpu
from jax.experimental.pallas import tpu_sc as plsc

N = 10000
E = 320000
NEX = 100000
NC, NS, L = 2, 16, 16
NW = NC * NS

KCH = 512
NCH = 20
NBUF = 4
E_PAD = NW * NCH * KCH
Z_SH_ROWS = 10240
DUMMY = N
ACC_ROWS = 10112
RPS = ACC_ROWS // NS

EX_T = 3200
EX_PAD = NW * EX_T

def _f32(*shape):
    return jax.ShapeDtypeStruct(shape, jnp.float32)


@functools.cache
def _mesh():
    return plsc.VectorSubcoreMesh(
        core_axis_name="c", subcore_axis_name="s",
        num_cores=NC, num_subcores=NS)


_SC_PARAMS = pltpu.CompilerParams(
    use_tc_tiling_on_sc=False, needs_layout_passes=False)


@functools.cache
def _make_sc_degree():
    @functools.partial(
        pl.kernel,
        out_type=[_f32(ACC_ROWS, 16), _f32(ACC_ROWS, 16)],
        mesh=_mesh(),
        compiler_params=_SC_PARAMS,
        scratch_types=[
            pltpu.VMEM((NCH, KCH), jnp.int32),
            pltpu.VMEM((KCH, 16), jnp.float32),
            pltpu.VMEM_SHARED((ACC_ROWS, 16), jnp.float32),
            pltpu.SemaphoreType.DMA,
        ],
    )
    def k(dst_hbm, ones_hbm, zeros_hbm, p0_hbm, p1_hbm, dstv, onesv, acc, sem):
        cid = lax.axis_index("c")
        sid = lax.axis_index("s")
        wid = cid * NS + sid
        rs = pl.ds(sid * RPS, RPS)
        pltpu.sync_copy(dst_hbm.at[wid], dstv)
        pltpu.sync_copy(ones_hbm, onesv)
        pltpu.sync_copy(zeros_hbm.at[rs], acc.at[rs])
        plsc.subcore_barrier()

        hs = [pltpu.async_copy(onesv, acc.at[dstv.at[c]], sem, add=True)
              for c in range(NCH)]
        for h in hs:
            h.wait()

        plsc.subcore_barrier()

        @pl.when(cid == 0)
        def _():
            pltpu.sync_copy(acc.at[rs], p0_hbm.at[rs])

        @pl.when(cid == 1)
        def _():
            pltpu.sync_copy(acc.at[rs], p1_hbm.at[rs])

    return k


@functools.cache
def _make_sc_aggregate(F):
    @functools.partial(
        pl.kernel,
        out_type=[_f32(ACC_ROWS, F), _f32(ACC_ROWS, F)],
        mesh=_mesh(),
        compiler_params=_SC_PARAMS,
        scratch_types=[
            pltpu.VMEM((NCH, KCH), jnp.int32),
            pltpu.VMEM((NCH, KCH), jnp.int32),
            [pltpu.VMEM((KCH, F), jnp.float32)] * NBUF,
            pltpu.VMEM_SHARED((Z_SH_ROWS, F), jnp.float32),
            pltpu.VMEM_SHARED((ACC_ROWS, F), jnp.float32),
            [pltpu.SemaphoreType.DMA] * NBUF,
            [pltpu.SemaphoreType.DMA] * NBUF,
        ],
    )
    def k(src_hbm, dst_hbm, z_hbm, zeros_hbm, p0_hbm, p1_hbm,
          srcv, dstv, bufs, z_sh, acc, gsems, ssems):
        cid = lax.axis_index("c")
        sid = lax.axis_index("s")
        wid = cid * NS + sid
        rs = pl.ds(sid * RPS, RPS)
        pltpu.sync_copy(src_hbm.at[wid], srcv)
        pltpu.sync_copy(dst_hbm.at[wid], dstv)
        pltpu.sync_copy(zeros_hbm.at[rs], acc.at[rs])

        @pl.when(sid < NS - 1)
        def _():
            zs = pl.ds(sid * 640, 640)
            pltpu.sync_copy(z_hbm.at[zs], z_sh.at[zs])

        @pl.when(sid == NS - 1)
        def _():
            zs = pl.ds((NS - 1) * 640, 400)
            pltpu.sync_copy(z_hbm.at[zs], z_sh.at[zs])

        plsc.subcore_barrier()

        def fire_gather(c):
            return pltpu.async_copy(
                z_sh.at[srcv.at[c]], bufs[c % NBUF], gsems[c % NBUF])

        def fire_scatter(c):
            return pltpu.async_copy(
                bufs[c % NBUF], acc.at[dstv.at[c]], ssems[c % NBUF],
                add=True)

        depth = 2
        gh = {c: fire_gather(c) for c in range(depth)}
        sh = {}
        for c in range(NCH):
            gh[c].wait()
            sh[c] = fire_scatter(c)
            nxt = c + depth
            if nxt < NCH:
                if nxt >= NBUF:
                    sh[nxt - NBUF].wait()
                    del sh[nxt - NBUF]
                gh[nxt] = fire_gather(nxt)
        for c in sorted(sh):
            sh[c].wait()

        plsc.subcore_barrier()

        @pl.when(cid == 0)
        def _():
            pltpu.sync_copy(acc.at[rs], p0_hbm.at[rs])

        @pl.when(cid == 1)
        def _():
            pltpu.sync_copy(acc.at[rs], p1_hbm.at[rs])

    return k


@functools.cache
def _make_sc_score():
    @functools.partial(
        pl.kernel,
        out_type=_f32(EX_PAD),
        mesh=_mesh(),
        compiler_params=_SC_PARAMS,
        scratch_types=[
            pltpu.VMEM((N,), jnp.float32),
            pltpu.VMEM((N,), jnp.float32),
            pltpu.VMEM((EX_T,), jnp.int32),
            pltpu.VMEM((EX_T,), jnp.int32),
            pltpu.VMEM((EX_T,), jnp.float32),
        ],
    )
    def k(scores_hbm, ex0_hbm, ex1_hbm, out_hbm, siv, sjv, e0v, e1v, outv):
        cid = lax.axis_index("c")
        sid = lax.axis_index("s")
        wid = cid * NS + sid
        pltpu.sync_copy(scores_hbm.at[0], siv)
        pltpu.sync_copy(scores_hbm.at[1], sjv)
        pltpu.sync_copy(ex0_hbm.at[wid], e0v)
        pltpu.sync_copy(ex1_hbm.at[wid], e1v)

        @pl.loop(0, EX_T, step=L)
        def _(i):
            i0 = e0v.at[pl.ds(i, L)][...]
            i1 = e1v.at[pl.ds(i, L)][...]
            a = plsc.load_gather(siv, [i0])
            b = plsc.load_gather(sjv, [i1])
            outv.at[pl.ds(i, L)][...] = 1.0 / (1.0 + jnp.exp(-(a + b)))

        pltpu.sync_copy(outv, out_hbm.at[pl.ds(wid * EX_T, EX_T)])

    return k


def _tc1_body(x_ref, w1_ref, d0_ref, d1_ref, z1_ref, y_ref):
    deg = d0_ref[0:N, :] + d1_ref[0:N, :] + 1.0
    y16 = lax.rsqrt(deg)
    xw = jnp.dot(x_ref[...], w1_ref[...], preferred_element_type=jnp.float32,
                 precision=lax.Precision.HIGHEST)
    z1_ref[...] = xw * y16
    y_ref[...] = y16


def _tc1(x, w1, d0, d1):
    return pl.pallas_call(
        _tc1_body, out_shape=[_f32(N, 16), _f32(N, 16)]
    )(x, w1, d0, d1)


def _tc2_body(y_ref, z1_ref, p0_ref, p1_ref, b1_ref, w2_ref, z2_ref):
    y16 = y_ref[...]
    h1 = jnp.maximum(y16 * (p0_ref[0:N, :] + p1_ref[0:N, :] + z1_ref[...])
                     + b1_ref[...], 0.0)
    xw2 = jnp.dot(h1, w2_ref[...], preferred_element_type=jnp.float32,
                  precision=lax.Precision.HIGHEST)
    y32 = jnp.concatenate([y16, y16], axis=1)
    z2_ref[...] = xw2 * y32


def _tc2(y16, z1, p0, p1, b1, w2):
    return pl.pallas_call(_tc2_body, out_shape=_f32(N, 32))(
        y16, z1, p0, p1, b1, w2)


def _tc3_body(y_ref, z2_ref, q0_ref, q1_ref, b2_ref, wfc_ref, bfc_ref,
              scores_ref):
    y16 = y_ref[...]
    y32 = jnp.concatenate([y16, y16], axis=1)
    h2 = y32 * (q0_ref[0:N, :] + q1_ref[0:N, :] + z2_ref[...]) + b2_ref[...]
    wi = wfc_ref[0:32, :]
    wj = wfc_ref[32:64, :]
    dn = (((0,), (1,)), ((), ()))
    si = lax.dot_general(wi, h2, dn, preferred_element_type=jnp.float32,
                         precision=lax.Precision.HIGHEST)
    sj = lax.dot_general(wj, h2, dn, preferred_element_type=jnp.float32,
                         precision=lax.Precision.HIGHEST)
    scores_ref[0:1, :] = si + bfc_ref[...]
    scores_ref[1:2, :] = sj


def _tc3(y16, z2, q0, q1, b2, wfc, bfc):
    return pl.pallas_call(_tc3_body, out_shape=_f32(2, N))(
        y16, z2, q0, q1, b2, wfc, bfc)


def kernel(x, edge_index, examples, W1, b1, W2, b2, Wfc, bfc):
    src = edge_index[0].astype(jnp.int32)
    dst = edge_index[1].astype(jnp.int32)
    epad = E_PAD - E
    src_p = jnp.concatenate(
        [src, jnp.zeros((epad,), jnp.int32)]).reshape(NW, NCH, KCH)
    dst_p = jnp.concatenate(
        [dst, jnp.full((epad,), DUMMY, jnp.int32)]).reshape(NW, NCH, KCH)
    xpad = EX_PAD - NEX
    ex0 = jnp.concatenate(
        [examples[:, 0].astype(jnp.int32), jnp.zeros((xpad,), jnp.int32)]
    ).reshape(NW, EX_T)
    ex1 = jnp.concatenate(
        [examples[:, 1].astype(jnp.int32), jnp.zeros((xpad,), jnp.int32)]
    ).reshape(NW, EX_T)

    ones16 = jnp.ones((KCH, 16), jnp.float32)
    zeros16 = jnp.zeros((ACC_ROWS, 16), jnp.float32)
    zeros32 = jnp.zeros((ACC_ROWS, 32), jnp.float32)

    d0, d1 = _make_sc_degree()(dst_p, ones16, zeros16)
    z1, y16 = _tc1(x, W1, d0, d1)
    p0, p1 = _make_sc_aggregate(16)(src_p, dst_p, z1, zeros16)
    z2 = _tc2(y16, z1, p0, p1, b1.reshape(1, 16), W2)
    q0, q1 = _make_sc_aggregate(32)(src_p, dst_p, z2, zeros32)
    scores = _tc3(y16, z2, q0, q1, b2.reshape(1, 32), Wfc,
                  bfc.reshape(1, 1))
    out = _make_sc_score()(scores, ex0, ex1)
    return out[:NEX]
